# baseline (speedup 1.0000x reference)
import functools

import jax
import jax.numpy as jnp
from jax import lax
from jax.experimental import pallas as pl
from jax.experimental.pallas import tpu as pltpu
from jax.experimental.pallas import tpu_sc as plsc

N = 10000
M = 320000
D = 128
L = 16
NC = 2
NS = 16
NW = NC * NS
C = 320
C_LAST = N - (NW - 1) * C
H = C // 2
CH = 184
CHB = 192
NB = M // L
NEG = float("-inf")
NJ = D // L
UR = 8


def _lower_bound3(seg_hbm, probe_v, sems, targets):
    def body(_, st):
        los, his, dones, anss = st
        mids = []
        cps = []
        for q in range(3):
            midb = lax.div(los[q] + his[q], jnp.int32(2))
            mids.append(midb)
            cp = pltpu.make_async_copy(
                seg_hbm.at[pl.ds(midb * L, L)],
                probe_v.at[pl.ds(q * L, L)], sems[q])
            cp.start()
            cps.append(cp)
        nlo, nhi, ndone, nans = [], [], [], []
        for q in range(3):
            cps[q].wait()
            v = probe_v[pl.ds(q * L, L)]
            live = jnp.logical_and(los[q] < his[q],
                                   jnp.logical_not(dones[q]))
            c = jnp.sum(jnp.where(v < targets[q],
                                  jnp.float32(1.0), jnp.float32(0.0))
                        ).astype(jnp.int32)
            nlo.append(jnp.where(jnp.logical_and(live, c == L),
                                 mids[q] + 1, los[q]))
            nhi.append(jnp.where(jnp.logical_and(live, c == 0),
                                 mids[q], his[q]))
            hit = jnp.logical_and(live, jnp.logical_and(c > 0, c < L))
            nans.append(jnp.where(hit, mids[q] * L + c, anss[q]))
            ndone.append(jnp.logical_or(dones[q], hit))
        return tuple(nlo), tuple(nhi), tuple(ndone), tuple(nans)

    z3 = (jnp.int32(0),) * 3
    los, his, dones, anss = lax.fori_loop(
        0, 16, body,
        (z3, (jnp.int32(NB),) * 3, (jnp.bool_(False),) * 3, z3))
    return [jnp.where(dones[q], anss[q], los[q] * L) for q in range(3)]


def _chunk_start(rs, ci):
    cs = rs + ci * CH
    a8 = pl.multiple_of(
        jnp.minimum(lax.bitwise_and(cs, jnp.int32(~7)), jnp.int32(M - CHB)),
        8)
    return cs, a8


def _pass(seg_hbm, lane_hbm, out_hbm, segc_v, rows_v, acc_v, cnt_v,
          sem_r0, sem_r1, sem_s0, sem_s1, w, base, rs, re,
          do_prime=True, nxt_rs=None, nxt_re=None):
    zf = jnp.zeros((L,), jnp.float32)
    nrows = re - rs

    def dma_pair_at(rs_, ci, buf, sem_r, sem_s):
        _, a8 = _chunk_start(rs_, ci)
        cp_r = pltpu.make_async_copy(
            lane_hbm.at[pl.ds(a8, CHB), :], rows_v.at[buf], sem_r)
        cp_s = pltpu.make_async_copy(
            seg_hbm.at[pl.ds(a8, CHB)],
            segc_v.at[pl.ds(buf * (CHB + L), CHB)], sem_s)
        return cp_r, cp_s

    def dma_pair(ci, buf, sem_r, sem_s):
        return dma_pair_at(rs, ci, buf, sem_r, sem_s)

    if do_prime:
        @pl.when(nrows > 0)
        def _prime():
            cp_r, cp_s = dma_pair(jnp.int32(0), 0, sem_r0, sem_s0)
            cp_r.start()
            cp_s.start()

    def zero_row(i, _):
        for j in range(2 * NJ):
            acc_v[i, pl.ds(L * j, L)] = zf
        cnt_v[i, :] = zf
        return 0

    lax.fori_loop(0, H, zero_row, 0)

    @pl.when(nrows > 0)
    def _process():
        nch = lax.div(nrows + (CH - 1), jnp.int32(CH))
        neg_v = jnp.full((L,), NEG, jnp.float32)
        one_v = jnp.ones((L,), jnp.float32)

        def flush_to(loc, cntv, accs):
            for j in range(NJ):
                acc_v[loc, pl.ds(L * j, L)] = accs[j]
                acc_v[loc, pl.ds(D + L * j, L)] = accs[NJ + j]
            cnt_v[loc, :] = cntv

        def step(buf, roff, s, rc):
            cur = rc[0]
            cntv = rc[1]
            accs = rc[2:]
            flush = s != cur

            @pl.when(jnp.logical_and(flush, cur >= 0))
            def _flush():
                flush_to(cur - base, cntv, accs)

            new_mx = []
            new_sm = []
            for j in range(NJ):
                v = rows_v[buf, roff, pl.ds(L * j, L)]
                new_mx.append(
                    jnp.maximum(jnp.where(flush, neg_v, accs[j]), v))
                new_sm.append(jnp.where(flush, zf, accs[NJ + j]) + v)
            new_cnt = jnp.where(flush, zf, cntv) + one_v
            return (s, new_cnt) + tuple(new_mx) + tuple(new_sm)

        def process_chunk(ci, buf, carry):
            cs, a8 = _chunk_start(rs, ci)
            nr = jnp.clip(re - cs, 0, CH)
            off = cs - a8

            soff = buf * (CHB + L)

            def run_fast(rc, r0, n):
                mx = list(rc[2:2 + NJ])
                sm = list(rc[2 + NJ:])
                for k in range(n):
                    for j in range(NJ):
                        v = rows_v[buf, r0 + k, pl.ds(L * j, L)]
                        mx[j] = jnp.maximum(mx[j], v)
                        sm[j] = sm[j] + v
                return (rc[0], rc[1] + jnp.float32(n)) + tuple(
                    mx) + tuple(sm)

            def half_body(rc, sv, r0, k0):
                nf = sv[k0 + UR - 1] != rc[0]

                def fast(rc):
                    return run_fast(rc, r0, UR)

                def slow(rc):
                    for k in range(UR):
                        rc = step(buf, r0 + k, sv[k0 + k], rc)
                    return rc

                return lax.cond(nf, slow, fast, rc)

            def group_body(g, rc):
                r0 = off + g * (2 * UR)
                sv = segc_v[pl.ds(soff + r0, L)]
                nf = sv[2 * UR - 1] != rc[0]

                def fast(rc):
                    return run_fast(rc, r0, 2 * UR)

                def slow(rc):
                    rc = half_body(rc, sv, r0, 0)
                    return half_body(rc, sv, r0 + UR, UR)

                return lax.cond(nf, slow, fast, rc)

            ng = lax.div(nr, jnp.int32(2 * UR))
            carry = lax.fori_loop(0, ng, group_body, carry)

            def tail_body(r, rc):
                roff = off + r
                s = segc_v[pl.ds(soff + roff, L)][0]
                return step(buf, roff, s, rc)

            return lax.fori_loop(ng * (2 * UR), nr, tail_body, carry)

        def chunk2_body(c2, carry):
            k0 = c2 * 2

            @pl.when(k0 + 1 < nch)
            def _issue1():
                cp_r, cp_s = dma_pair(k0 + 1, 1, sem_r1, sem_s1)
                cp_r.start()
                cp_s.start()

            cp_r, cp_s = dma_pair(k0, 0, sem_r0, sem_s0)
            cp_r.wait()
            cp_s.wait()
            carry = process_chunk(k0, 0, carry)

            @pl.when(k0 + 2 < nch)
            def _issue2():
                cp_r, cp_s = dma_pair(k0 + 2, 0, sem_r0, sem_s0)
                cp_r.start()
                cp_s.start()

            @pl.when(k0 + 1 < nch)
            def _wait1():
                cp_r, cp_s = dma_pair(k0 + 1, 1, sem_r1, sem_s1)
                cp_r.wait()
                cp_s.wait()

            return process_chunk(k0 + 1, 1, carry)

        init = (jnp.int32(-1), zf) + tuple(
            jnp.full((L,), NEG, jnp.float32) for _ in range(NJ)) + tuple(
            jnp.zeros((L,), jnp.float32) for _ in range(NJ))
        nc2 = lax.div(nch + 1, jnp.int32(2))
        fin = lax.fori_loop(0, nc2, chunk2_body, init)

        cur = fin[0]

        @pl.when(cur >= 0)
        def _final_flush():
            flush_to(cur - base, fin[1], fin[2:])

    if nxt_rs is not None:
        @pl.when(nxt_re - nxt_rs > 0)
        def _prime_next():
            cp_r, cp_s = dma_pair_at(nxt_rs, jnp.int32(0), 0,
                                     sem_r0, sem_s0)
            cp_r.start()
            cp_s.start()

    def div_row(i, _):
        cv = cnt_v[i, :]
        inv = 1.0 / jnp.maximum(cv, 1.0)
        for j in range(NJ):
            acc_v[i, pl.ds(D + L * j, L)] = acc_v[i, pl.ds(D + L * j, L)] * inv
        return 0

    lax.fori_loop(0, H, div_row, 0)

    @pl.when(w < NW - 1)
    def _out_full():
        pltpu.sync_copy(acc_v, out_hbm.at[pl.ds(base, H), :])

    @pl.when(jnp.logical_and(w == NW - 1, base == (NW - 1) * C))
    def _out_last():
        pltpu.sync_copy(acc_v.at[pl.ds(0, C_LAST), :],
                        out_hbm.at[pl.ds(base, C_LAST), :])


def _body(seg_hbm, lane_hbm, out_hbm, probe_v, segc_v, rows_v, acc_v, cnt_v,
          sem_r0, sem_r1, sem_s0, sem_s1):
    cid = lax.axis_index("c")
    sid = lax.axis_index("s")
    w = sid * NC + cid
    base = w * C

    r0, r1, r2 = _lower_bound3(
        seg_hbm, probe_v, (sem_r0, sem_r1, sem_s0),
        (base, base + H, base + 2 * H))

    _pass(seg_hbm, lane_hbm, out_hbm, segc_v, rows_v, acc_v, cnt_v,
          sem_r0, sem_r1, sem_s0, sem_s1, w, base, r0, r1,
          do_prime=True, nxt_rs=r1, nxt_re=r2)
    _pass(seg_hbm, lane_hbm, out_hbm, segc_v, rows_v, acc_v, cnt_v,
          sem_r0, sem_r1, sem_s0, sem_s1, w, base + H, r1, r2,
          do_prime=False)


@jax.jit
def _agg(seg, lane):
    mesh = plsc.VectorSubcoreMesh(core_axis_name="c", subcore_axis_name="s")
    return pl.kernel(
        _body,
        out_type=jax.ShapeDtypeStruct((N, 2 * D), jnp.float32),
        mesh=mesh,
        compiler_params=pltpu.CompilerParams(needs_layout_passes=False),
        scratch_types=[
            pltpu.VMEM((3 * L,), jnp.int32),
            pltpu.VMEM((2 * (CHB + L),), jnp.int32),
            pltpu.VMEM((2, CHB, D), jnp.float32),
            pltpu.VMEM((H, 2 * D), jnp.float32),
            pltpu.VMEM((H, L), jnp.float32),
            pltpu.SemaphoreType.DMA,
            pltpu.SemaphoreType.DMA,
            pltpu.SemaphoreType.DMA,
            pltpu.SemaphoreType.DMA,
        ],
    )(seg, lane)


def kernel(obs_encoding, lane_encoding, same_obs_mask):
    seg = same_obs_mask.reshape(M)
    return _agg(seg, lane_encoding)


# speedup vs baseline: 1.1404x; 1.1404x over previous
import functools

import jax
import jax.numpy as jnp
from jax import lax
from jax.experimental import pallas as pl
from jax.experimental.pallas import tpu as pltpu
from jax.experimental.pallas import tpu_sc as plsc

N = 10000
M = 320000
D = 128
L = 16
NC = 2
NS = 16
NW = NC * NS
C = 320
C_LAST = N - (NW - 1) * C
H = C // 2
CH = 184
CHB = 192
NB = M // L
NEG = float("-inf")
NJ = D // L
UR = 8


def _lower_bound3(seg_hbm, probe_v, sems, targets):
    def body(_, st):
        los, his, dones, anss = st
        mids = []
        cps = []
        for q in range(3):
            midb = lax.div(los[q] + his[q], jnp.int32(2))
            mids.append(midb)
            cp = pltpu.make_async_copy(
                seg_hbm.at[pl.ds(midb * L, L)],
                probe_v.at[pl.ds(q * L, L)], sems[q])
            cp.start()
            cps.append(cp)
        nlo, nhi, ndone, nans = [], [], [], []
        for q in range(3):
            cps[q].wait()
            v = probe_v[pl.ds(q * L, L)]
            live = jnp.logical_and(los[q] < his[q],
                                   jnp.logical_not(dones[q]))
            c = jnp.sum(jnp.where(v < targets[q],
                                  jnp.float32(1.0), jnp.float32(0.0))
                        ).astype(jnp.int32)
            nlo.append(jnp.where(jnp.logical_and(live, c == L),
                                 mids[q] + 1, los[q]))
            nhi.append(jnp.where(jnp.logical_and(live, c == 0),
                                 mids[q], his[q]))
            hit = jnp.logical_and(live, jnp.logical_and(c > 0, c < L))
            nans.append(jnp.where(hit, mids[q] * L + c, anss[q]))
            ndone.append(jnp.logical_or(dones[q], hit))
        return tuple(nlo), tuple(nhi), tuple(ndone), tuple(nans)

    z3 = (jnp.int32(0),) * 3
    los, his, dones, anss = lax.fori_loop(
        0, 16, body,
        (z3, (jnp.int32(NB),) * 3, (jnp.bool_(False),) * 3, z3))
    return [jnp.where(dones[q], anss[q], los[q] * L) for q in range(3)]


def _chunk_start(rs, ci):
    cs = rs + ci * CH
    a8 = pl.multiple_of(
        jnp.minimum(lax.bitwise_and(cs, jnp.int32(~7)), jnp.int32(M - CHB)),
        8)
    return cs, a8


def _pass(seg_hbm, lane_hbm, out_hbm, segc_v, rows_v, acc_v, cnt_v,
          sem_r0, sem_r1, sem_s0, sem_s1, w, base, rs, re,
          do_prime=True, nxt_rs=None, nxt_re=None):
    zf = jnp.zeros((L,), jnp.float32)
    nrows = re - rs

    def dma_pair_at(rs_, ci, buf, sem_r, sem_s):
        _, a8 = _chunk_start(rs_, ci)
        cp_r = pltpu.make_async_copy(
            lane_hbm.at[pl.ds(a8, CHB), :], rows_v.at[buf], sem_r)
        cp_s = pltpu.make_async_copy(
            seg_hbm.at[pl.ds(a8, CHB)],
            segc_v.at[pl.ds(buf * (CHB + L), CHB)], sem_s)
        return cp_r, cp_s

    def dma_pair(ci, buf, sem_r, sem_s):
        return dma_pair_at(rs, ci, buf, sem_r, sem_s)

    if do_prime:
        @pl.when(nrows > 0)
        def _prime():
            cp_r, cp_s = dma_pair(jnp.int32(0), 0, sem_r0, sem_s0)
            cp_r.start()
            cp_s.start()

    def zero_row(i, _):
        for j in range(2 * NJ):
            acc_v[i, pl.ds(L * j, L)] = zf
        cnt_v[i, :] = zf
        return 0

    lax.fori_loop(0, H, zero_row, 0)

    @pl.when(nrows > 0)
    def _process():
        nch = lax.div(nrows + (CH - 1), jnp.int32(CH))
        neg_v = jnp.full((L,), NEG, jnp.float32)
        one_v = jnp.ones((L,), jnp.float32)

        def flush_to(loc, cntv, accs):
            for j in range(NJ):
                acc_v[loc, pl.ds(L * j, L)] = accs[j]
                acc_v[loc, pl.ds(D + L * j, L)] = accs[NJ + j]
            cnt_v[loc, :] = cntv

        def step(buf, roff, s, rc):
            cur = rc[0]
            cntv = rc[1]
            accs = rc[2:]
            flush = s != cur

            @pl.when(jnp.logical_and(flush, cur >= 0))
            def _flush():
                flush_to(cur - base, cntv, accs)

            new_mx = []
            new_sm = []
            for j in range(NJ):
                v = rows_v[buf, roff, pl.ds(L * j, L)]
                new_mx.append(
                    jnp.maximum(jnp.where(flush, neg_v, accs[j]), v))
                new_sm.append(jnp.where(flush, zf, accs[NJ + j]) + v)
            new_cnt = jnp.where(flush, zf, cntv) + one_v
            return (s, new_cnt) + tuple(new_mx) + tuple(new_sm)

        def process_chunk(ci, buf, carry):
            cs, a8 = _chunk_start(rs, ci)
            nr = jnp.clip(re - cs, 0, CH)
            off = cs - a8

            soff = buf * (CHB + L)

            def run_fast(rc, r0, n):
                mx = list(rc[2:2 + NJ])
                sm = list(rc[2 + NJ:])
                for k in range(n):
                    for j in range(NJ):
                        v = rows_v[buf, r0 + k, pl.ds(L * j, L)]
                        mx[j] = jnp.maximum(mx[j], v)
                        sm[j] = sm[j] + v
                return (rc[0], rc[1] + jnp.float32(n)) + tuple(
                    mx) + tuple(sm)

            def group_body(g, rc):
                r0 = off + g * UR
                sv = segc_v[pl.ds(soff + r0, L)]
                nf = sv[UR - 1] != rc[0]

                def fast(rc):
                    return run_fast(rc, r0, UR)

                def slow(rc):
                    for k in range(UR):
                        rc = step(buf, r0 + k, sv[k], rc)
                    return rc

                return lax.cond(nf, slow, fast, rc)

            ng = lax.div(nr, jnp.int32(UR))
            carry = lax.fori_loop(0, ng, group_body, carry)

            def tail_body(r, rc):
                roff = off + r
                s = segc_v[pl.ds(soff + roff, L)][0]
                return step(buf, roff, s, rc)

            return lax.fori_loop(ng * UR, nr, tail_body, carry)

        def chunk2_body(c2, carry):
            k0 = c2 * 2

            @pl.when(k0 + 1 < nch)
            def _issue1():
                cp_r, cp_s = dma_pair(k0 + 1, 1, sem_r1, sem_s1)
                cp_r.start()
                cp_s.start()

            cp_r, cp_s = dma_pair(k0, 0, sem_r0, sem_s0)
            cp_r.wait()
            cp_s.wait()
            carry = process_chunk(k0, 0, carry)

            @pl.when(k0 + 2 < nch)
            def _issue2():
                cp_r, cp_s = dma_pair(k0 + 2, 0, sem_r0, sem_s0)
                cp_r.start()
                cp_s.start()

            @pl.when(k0 + 1 < nch)
            def _wait1():
                cp_r, cp_s = dma_pair(k0 + 1, 1, sem_r1, sem_s1)
                cp_r.wait()
                cp_s.wait()

            return process_chunk(k0 + 1, 1, carry)

        init = (jnp.int32(-1), zf) + tuple(
            jnp.full((L,), NEG, jnp.float32) for _ in range(NJ)) + tuple(
            jnp.zeros((L,), jnp.float32) for _ in range(NJ))
        nc2 = lax.div(nch + 1, jnp.int32(2))
        fin = lax.fori_loop(0, nc2, chunk2_body, init)

        cur = fin[0]

        @pl.when(cur >= 0)
        def _final_flush():
            flush_to(cur - base, fin[1], fin[2:])

    if nxt_rs is not None:
        @pl.when(nxt_re - nxt_rs > 0)
        def _prime_next():
            cp_r, cp_s = dma_pair_at(nxt_rs, jnp.int32(0), 0,
                                     sem_r0, sem_s0)
            cp_r.start()
            cp_s.start()

    def div_row(i, _):
        cv = cnt_v[i, :]
        inv = 1.0 / jnp.maximum(cv, 1.0)
        for j in range(NJ):
            acc_v[i, pl.ds(D + L * j, L)] = acc_v[i, pl.ds(D + L * j, L)] * inv
        return 0

    lax.fori_loop(0, H, div_row, 0)

    @pl.when(w < NW - 1)
    def _out_full():
        pltpu.sync_copy(acc_v, out_hbm.at[pl.ds(base, H), :])

    @pl.when(jnp.logical_and(w == NW - 1, base == (NW - 1) * C))
    def _out_last():
        pltpu.sync_copy(acc_v.at[pl.ds(0, C_LAST), :],
                        out_hbm.at[pl.ds(base, C_LAST), :])


def _body(seg_hbm, lane_hbm, out_hbm, probe_v, segc_v, rows_v, acc_v, cnt_v,
          sem_r0, sem_r1, sem_s0, sem_s1):
    cid = lax.axis_index("c")
    sid = lax.axis_index("s")
    w = sid * NC + cid
    base = w * C

    r0, r1, r2 = _lower_bound3(
        seg_hbm, probe_v, (sem_r0, sem_r1, sem_s0),
        (base, base + H, base + 2 * H))

    _pass(seg_hbm, lane_hbm, out_hbm, segc_v, rows_v, acc_v, cnt_v,
          sem_r0, sem_r1, sem_s0, sem_s1, w, base, r0, r1,
          do_prime=True, nxt_rs=r1, nxt_re=r2)
    _pass(seg_hbm, lane_hbm, out_hbm, segc_v, rows_v, acc_v, cnt_v,
          sem_r0, sem_r1, sem_s0, sem_s1, w, base + H, r1, r2,
          do_prime=False)


@jax.jit
def _agg(seg, lane):
    mesh = plsc.VectorSubcoreMesh(core_axis_name="c", subcore_axis_name="s")
    return pl.kernel(
        _body,
        out_type=jax.ShapeDtypeStruct((N, 2 * D), jnp.float32),
        mesh=mesh,
        compiler_params=pltpu.CompilerParams(needs_layout_passes=False),
        scratch_types=[
            pltpu.VMEM((3 * L,), jnp.int32),
            pltpu.VMEM((2 * (CHB + L),), jnp.int32),
            pltpu.VMEM((2, CHB, D), jnp.float32),
            pltpu.VMEM((H, 2 * D), jnp.float32),
            pltpu.VMEM((H, L), jnp.float32),
            pltpu.SemaphoreType.DMA,
            pltpu.SemaphoreType.DMA,
            pltpu.SemaphoreType.DMA,
            pltpu.SemaphoreType.DMA,
        ],
    )(seg, lane)


def kernel(obs_encoding, lane_encoding, same_obs_mask):
    seg = same_obs_mask.reshape(M)
    return _agg(seg, lane_encoding)


# speedup vs baseline: 1.1407x; 1.0002x over previous
import functools

import jax
import jax.numpy as jnp
from jax import lax
from jax.experimental import pallas as pl
from jax.experimental.pallas import tpu as pltpu
from jax.experimental.pallas import tpu_sc as plsc

N = 10000
M = 320000
D = 128
L = 16
NC = 2
NS = 16
NW = NC * NS
C = 320
C_LAST = N - (NW - 1) * C
H = C // 2
CH = 184
CHB = 192
NB = M // L
NEG = float("-inf")
NJ = D // L
UR = 8


def _lower_bound3(seg_hbm, probe_v, sems, targets):
    def body(_, st):
        los, his, dones, anss = st
        mids = []
        cps = []
        for q in range(3):
            midb = lax.div(los[q] + his[q], jnp.int32(2))
            mids.append(midb)
            cp = pltpu.make_async_copy(
                seg_hbm.at[pl.ds(midb * L, L)],
                probe_v.at[pl.ds(q * L, L)], sems[q])
            cp.start()
            cps.append(cp)
        nlo, nhi, ndone, nans = [], [], [], []
        for q in range(3):
            cps[q].wait()
            v = probe_v[pl.ds(q * L, L)]
            live = jnp.logical_and(los[q] < his[q],
                                   jnp.logical_not(dones[q]))
            c = jnp.sum(jnp.where(v < targets[q],
                                  jnp.float32(1.0), jnp.float32(0.0))
                        ).astype(jnp.int32)
            nlo.append(jnp.where(jnp.logical_and(live, c == L),
                                 mids[q] + 1, los[q]))
            nhi.append(jnp.where(jnp.logical_and(live, c == 0),
                                 mids[q], his[q]))
            hit = jnp.logical_and(live, jnp.logical_and(c > 0, c < L))
            nans.append(jnp.where(hit, mids[q] * L + c, anss[q]))
            ndone.append(jnp.logical_or(dones[q], hit))
        return tuple(nlo), tuple(nhi), tuple(ndone), tuple(nans)

    z3 = (jnp.int32(0),) * 3
    los, his, dones, anss = lax.fori_loop(
        0, 16, body,
        (z3, (jnp.int32(NB),) * 3, (jnp.bool_(False),) * 3, z3))
    return [jnp.where(dones[q], anss[q], los[q] * L) for q in range(3)]


def _chunk_start(rs, ci):
    cs = rs + ci * CH
    a8 = pl.multiple_of(
        jnp.minimum(lax.bitwise_and(cs, jnp.int32(~7)), jnp.int32(M - CHB)),
        8)
    return cs, a8


def _pass(seg_hbm, lane_hbm, out_hbm, segc_v, rows_v, acc_v, cnt_v,
          sem_r0, sem_r1, sem_s0, sem_s1, w, base, rs, re,
          do_prime=True, nxt_rs=None, nxt_re=None):
    zf = jnp.zeros((L,), jnp.float32)
    nrows = re - rs

    def dma_pair_at(rs_, ci, buf, sem_r, sem_s):
        _, a8 = _chunk_start(rs_, ci)
        cp_r = pltpu.make_async_copy(
            lane_hbm.at[pl.ds(a8, CHB), :], rows_v.at[buf], sem_r)
        cp_s = pltpu.make_async_copy(
            seg_hbm.at[pl.ds(a8, CHB)],
            segc_v.at[pl.ds(buf * (CHB + L), CHB)], sem_s)
        return cp_r, cp_s

    def dma_pair(ci, buf, sem_r, sem_s):
        return dma_pair_at(rs, ci, buf, sem_r, sem_s)

    if do_prime:
        @pl.when(nrows > 0)
        def _prime():
            cp_r, cp_s = dma_pair(jnp.int32(0), 0, sem_r0, sem_s0)
            cp_r.start()
            cp_s.start()

    def zero_row(i, _):
        for j in range(2 * NJ):
            acc_v[i, pl.ds(L * j, L)] = zf
        cnt_v[i, :] = zf
        return 0

    lax.fori_loop(0, H, zero_row, 0)

    @pl.when(nrows > 0)
    def _process():
        nch = lax.div(nrows + (CH - 1), jnp.int32(CH))
        neg_v = jnp.full((L,), NEG, jnp.float32)
        one_v = jnp.ones((L,), jnp.float32)

        def flush_to(loc, cntv, accs):
            for j in range(NJ):
                acc_v[loc, pl.ds(L * j, L)] = accs[j]
                acc_v[loc, pl.ds(D + L * j, L)] = accs[NJ + j]
            cnt_v[loc, :] = cntv

        def step(buf, roff, s, rc):
            cur = rc[0]
            cntv = rc[1]
            accs = rc[2:]
            flush = s != cur

            @pl.when(jnp.logical_and(flush, cur >= 0))
            def _flush():
                flush_to(cur - base, cntv, accs)

            new_mx = []
            new_sm = []
            for j in range(NJ):
                v = rows_v[buf, roff, pl.ds(L * j, L)]
                new_mx.append(
                    jnp.maximum(jnp.where(flush, neg_v, accs[j]), v))
                new_sm.append(jnp.where(flush, zf, accs[NJ + j]) + v)
            new_cnt = jnp.where(flush, zf, cntv) + one_v
            return (s, new_cnt) + tuple(new_mx) + tuple(new_sm)

        def process_chunk(ci, buf, carry):
            cs, a8 = _chunk_start(rs, ci)
            nr = jnp.clip(re - cs, 0, CH)
            off = cs - a8

            soff = buf * (CHB + L)

            def run_fast(rc, r0, n):
                mx = list(rc[2:2 + NJ])
                sm = list(rc[2 + NJ:])
                for k in range(n):
                    for j in range(NJ):
                        v = rows_v[buf, r0 + k, pl.ds(L * j, L)]
                        mx[j] = jnp.maximum(mx[j], v)
                        sm[j] = sm[j] + v
                return (rc[0], rc[1] + jnp.float32(n)) + tuple(
                    mx) + tuple(sm)

            def group_body(g, rc):
                r0 = off + g * UR
                sv = segc_v[pl.ds(soff + r0, L)]
                nf = sv[UR - 1] != rc[0]

                def fast(rc):
                    return run_fast(rc, r0, UR)

                def slow(rc):
                    for k in range(UR):
                        rc = step(buf, r0 + k, sv[k], rc)
                    return rc

                return lax.cond(nf, slow, fast, rc)

            ng = lax.div(nr, jnp.int32(UR))
            carry = plsc.parallel_loop(0, ng, carry=carry)(group_body)

            def tail_body(r, rc):
                roff = off + r
                s = segc_v[pl.ds(soff + roff, L)][0]
                return step(buf, roff, s, rc)

            return lax.fori_loop(ng * UR, nr, tail_body, carry)

        def chunk2_body(c2, carry):
            k0 = c2 * 2

            @pl.when(k0 + 1 < nch)
            def _issue1():
                cp_r, cp_s = dma_pair(k0 + 1, 1, sem_r1, sem_s1)
                cp_r.start()
                cp_s.start()

            cp_r, cp_s = dma_pair(k0, 0, sem_r0, sem_s0)
            cp_r.wait()
            cp_s.wait()
            carry = process_chunk(k0, 0, carry)

            @pl.when(k0 + 2 < nch)
            def _issue2():
                cp_r, cp_s = dma_pair(k0 + 2, 0, sem_r0, sem_s0)
                cp_r.start()
                cp_s.start()

            @pl.when(k0 + 1 < nch)
            def _wait1():
                cp_r, cp_s = dma_pair(k0 + 1, 1, sem_r1, sem_s1)
                cp_r.wait()
                cp_s.wait()

            return process_chunk(k0 + 1, 1, carry)

        init = (jnp.int32(-1), zf) + tuple(
            jnp.full((L,), NEG, jnp.float32) for _ in range(NJ)) + tuple(
            jnp.zeros((L,), jnp.float32) for _ in range(NJ))
        nc2 = lax.div(nch + 1, jnp.int32(2))
        fin = lax.fori_loop(0, nc2, chunk2_body, init)

        cur = fin[0]

        @pl.when(cur >= 0)
        def _final_flush():
            flush_to(cur - base, fin[1], fin[2:])

    if nxt_rs is not None:
        @pl.when(nxt_re - nxt_rs > 0)
        def _prime_next():
            cp_r, cp_s = dma_pair_at(nxt_rs, jnp.int32(0), 0,
                                     sem_r0, sem_s0)
            cp_r.start()
            cp_s.start()

    def div_row(i, _):
        cv = cnt_v[i, :]
        inv = 1.0 / jnp.maximum(cv, 1.0)
        for j in range(NJ):
            acc_v[i, pl.ds(D + L * j, L)] = acc_v[i, pl.ds(D + L * j, L)] * inv
        return 0

    lax.fori_loop(0, H, div_row, 0)

    @pl.when(w < NW - 1)
    def _out_full():
        pltpu.sync_copy(acc_v, out_hbm.at[pl.ds(base, H), :])

    @pl.when(jnp.logical_and(w == NW - 1, base == (NW - 1) * C))
    def _out_last():
        pltpu.sync_copy(acc_v.at[pl.ds(0, C_LAST), :],
                        out_hbm.at[pl.ds(base, C_LAST), :])


def _body(seg_hbm, lane_hbm, out_hbm, probe_v, segc_v, rows_v, acc_v, cnt_v,
          sem_r0, sem_r1, sem_s0, sem_s1):
    cid = lax.axis_index("c")
    sid = lax.axis_index("s")
    w = sid * NC + cid
    base = w * C

    r0, r1, r2 = _lower_bound3(
        seg_hbm, probe_v, (sem_r0, sem_r1, sem_s0),
        (base, base + H, base + 2 * H))

    _pass(seg_hbm, lane_hbm, out_hbm, segc_v, rows_v, acc_v, cnt_v,
          sem_r0, sem_r1, sem_s0, sem_s1, w, base, r0, r1,
          do_prime=True, nxt_rs=r1, nxt_re=r2)
    _pass(seg_hbm, lane_hbm, out_hbm, segc_v, rows_v, acc_v, cnt_v,
          sem_r0, sem_r1, sem_s0, sem_s1, w, base + H, r1, r2,
          do_prime=False)


@jax.jit
def _agg(seg, lane):
    mesh = plsc.VectorSubcoreMesh(core_axis_name="c", subcore_axis_name="s")
    return pl.kernel(
        _body,
        out_type=jax.ShapeDtypeStruct((N, 2 * D), jnp.float32),
        mesh=mesh,
        compiler_params=pltpu.CompilerParams(needs_layout_passes=False),
        scratch_types=[
            pltpu.VMEM((3 * L,), jnp.int32),
            pltpu.VMEM((2 * (CHB + L),), jnp.int32),
            pltpu.VMEM((2, CHB, D), jnp.float32),
            pltpu.VMEM((H, 2 * D), jnp.float32),
            pltpu.VMEM((H, L), jnp.float32),
            pltpu.SemaphoreType.DMA,
            pltpu.SemaphoreType.DMA,
            pltpu.SemaphoreType.DMA,
            pltpu.SemaphoreType.DMA,
        ],
    )(seg, lane)


def kernel(obs_encoding, lane_encoding, same_obs_mask):
    seg = same_obs_mask.reshape(M)
    return _agg(seg, lane_encoding)
